# baseline (speedup 1.0000x reference)
import functools

import jax
import jax.numpy as jnp
from jax.experimental import pallas as pl
from jax.experimental.pallas import tpu as pltpu


def _embed_kernel(x_ref, w_ref, add_ref, out_ref):
    acc = jnp.dot(x_ref[...], w_ref[...], preferred_element_type=jnp.float32)
    out_ref[...] = acc + add_ref[...]


@functools.partial(jax.jit, static_argnames=("tm", "tn", "interpret"))
def _embed(seq2d, W_tok, addend, tm=896, tn=2048, interpret=False):
    M, C = seq2d.shape
    D = W_tok.shape[1]
    grid = (D // tn, M // tm)
    out = pl.pallas_call(
        _embed_kernel,
        grid=grid,
        in_specs=[
            pl.BlockSpec((tm, C), lambda j, i: (i, 0)),
            pl.BlockSpec((C, tn), lambda j, i: (0, j)),
            pl.BlockSpec((tm, tn), lambda j, i: (0, j)),
        ],
        out_specs=pl.BlockSpec((tm, tn), lambda j, i: (i, j)),
        out_shape=jax.ShapeDtypeStruct((M, D), jnp.float32),
        compiler_params=pltpu.CompilerParams(
            dimension_semantics=("arbitrary", "arbitrary"),
        ),
        interpret=interpret,
    )(seq2d, W_tok, addend)
    return out


def kernel(sequence, W_tok, b_tok, pos_table):
    B, L, C = sequence.shape
    D = W_tok.shape[1]
    tm = 128 * L
    seq2d = sequence.reshape(B * L, C)
    addend = jnp.tile(pos_table + b_tok[None, :], (tm // L, 1))
    out = _embed(seq2d, W_tok, addend, tm=tm, tn=D)
    x = out.reshape(B, L, D)
    mask = jnp.ones((B, L), dtype=bool)
    return (x, mask)
